# baseline (speedup 1.0000x reference)
import functools

import jax
import jax.numpy as jnp
from jax import lax
from jax.experimental import pallas as pl
from jax.experimental.pallas import tpu as pltpu
from jax.experimental.pallas import tpu_sc as plsc

_B = 16384
_D = 64
_H1 = 256
_H2 = 128
_EPS = 1e-5
_BM = 2048


@functools.lru_cache(maxsize=1)
def _make_sc_gather():
    info = plsc.get_sparse_core_info()
    nc, ns = info.num_cores, info.num_subcores
    nw = nc * ns
    bpw = _B // nw
    mesh = plsc.VectorSubcoreMesh(core_axis_name="c", subcore_axis_name="s")

    @functools.partial(
        pl.kernel,
        mesh=mesh,
        compiler_params=pltpu.CompilerParams(needs_layout_passes=False),
        out_type=[
            jax.ShapeDtypeStruct((_B, _D), jnp.float32),
            jax.ShapeDtypeStruct((_B, _D), jnp.float32),
        ],
        scratch_types=[
            pltpu.VMEM((bpw,), jnp.int32),
            pltpu.VMEM((bpw, _D), jnp.float32),
            pltpu.SemaphoreType.DMA,
            pltpu.SemaphoreType.DMA,
            pltpu.SemaphoreType.DMA,
        ],
    )
    def gather_k(tid_hbm, aid_hbm, ttab_hbm, atab_hbm, t_out, a_out,
                 ids_v, rows_v, gsem, wsem_t, wsem_a):
        wid = lax.axis_index("s") * nc + lax.axis_index("c")
        base = wid * bpw
        lane = lax.iota(jnp.int32, 16)

        def gather_table(id_hbm, tab_hbm, out_hbm, wsem):
            pltpu.sync_copy(id_hbm.at[pl.ds(base, bpw)], ids_v)

            def fire16(g, carry):
                ids16 = ids_v[pl.ds(g * 16, 16)]
                for j in range(16):
                    rid = jnp.sum(jnp.where(lane == j, ids16, 0))
                    pltpu.async_copy(
                        tab_hbm.at[rid], rows_v.at[g * 16 + j], gsem)
                return carry

            lax.fori_loop(0, bpw // 16, fire16, 0)

            def drain(i, carry):
                pltpu.make_async_copy(
                    tab_hbm.at[0], rows_v.at[0], gsem).wait()
                return carry

            lax.fori_loop(0, bpw, drain, 0)
            return pltpu.async_copy(rows_v, out_hbm.at[pl.ds(base, bpw)], wsem)

        wt = gather_table(tid_hbm, ttab_hbm, t_out, wsem_t)
        wt.wait()
        wa = gather_table(aid_hbm, atab_hbm, a_out, wsem_a)
        wa.wait()

    return gather_k


def _stage1_body(t, a, genre, audio, temporal,
                 gw, gb, auw, aub, tew, teb,
                 w1t, w1a, w1g, w1au, w1te, b1,
                 y1, s1, ss1):
    i = pl.program_id(0)
    g = jax.nn.relu(jnp.dot(genre[...], gw[...],
                            preferred_element_type=jnp.float32) + gb[...])
    au = jax.nn.relu(jnp.dot(audio[...], auw[...],
                             preferred_element_type=jnp.float32) + aub[...])
    te = jax.nn.relu(jnp.dot(temporal[...], tew[...],
                             preferred_element_type=jnp.float32) + teb[...])
    y = (jnp.dot(t[...], w1t[...], preferred_element_type=jnp.float32)
         + jnp.dot(a[...], w1a[...], preferred_element_type=jnp.float32)
         + jnp.dot(g, w1g[...], preferred_element_type=jnp.float32)
         + jnp.dot(au, w1au[...], preferred_element_type=jnp.float32)
         + jnp.dot(te, w1te[...], preferred_element_type=jnp.float32)
         + b1[...])
    y1[...] = y

    @pl.when(i == 0)
    def _():
        s1[...] = jnp.zeros_like(s1)
        ss1[...] = jnp.zeros_like(ss1)

    s1[...] += jnp.sum(y, axis=0, keepdims=True)
    ss1[...] += jnp.sum(y * y, axis=0, keepdims=True)


def _stage2_body(y1, s1, ss1, g1, be1, w2, b2, y2, s2, ss2):
    i = pl.program_id(0)
    mu = s1[...] * (1.0 / _B)
    var = ss1[...] * (1.0 / _B) - mu * mu
    scale = g1[...] * lax.rsqrt(var + _EPS)
    h = jax.nn.relu((y1[...] - mu) * scale + be1[...])
    y = jnp.dot(h, w2[...], preferred_element_type=jnp.float32) + b2[...]
    y2[...] = y

    @pl.when(i == 0)
    def _():
        s2[...] = jnp.zeros_like(s2)
        ss2[...] = jnp.zeros_like(ss2)

    s2[...] += jnp.sum(y, axis=0, keepdims=True)
    ss2[...] += jnp.sum(y * y, axis=0, keepdims=True)


def _stage3_body(y2, s2, ss2, g2, be2, out):
    mu = s2[...] * (1.0 / _B)
    var = ss2[...] * (1.0 / _B) - mu * mu
    scale = g2[...] * lax.rsqrt(var + _EPS)
    out[...] = jax.nn.relu((y2[...] - mu) * scale + be2[...])


def _row_spec(bm, n):
    return pl.BlockSpec((bm, n), lambda i: (i, 0))


def _full_spec(m, n):
    return pl.BlockSpec((m, n), lambda i: (0, 0))


def _tower(t, a, genre, audio, temporal,
           genre_W, genre_b, audio_W, audio_b, temporal_W, temporal_b,
           W1, b1, g1, be1, W2, b2, g2, be2):
    nb = _B // _BM
    w1t, w1a, w1g, w1au, w1te = (W1[0:64], W1[64:128], W1[128:192],
                                 W1[192:256], W1[256:320])
    row2 = lambda v: v.reshape(1, -1)

    y1, s1, ss1 = pl.pallas_call(
        _stage1_body,
        grid=(nb,),
        in_specs=[
            _row_spec(_BM, _D), _row_spec(_BM, _D), _row_spec(_BM, 32),
            _row_spec(_BM, 16), _row_spec(_BM, 8),
            _full_spec(32, _D), _full_spec(1, _D),
            _full_spec(16, _D), _full_spec(1, _D),
            _full_spec(8, _D), _full_spec(1, _D),
            _full_spec(_D, _H1), _full_spec(_D, _H1), _full_spec(_D, _H1),
            _full_spec(_D, _H1), _full_spec(_D, _H1), _full_spec(1, _H1),
        ],
        out_specs=[
            _row_spec(_BM, _H1), _full_spec(1, _H1), _full_spec(1, _H1),
        ],
        out_shape=[
            jax.ShapeDtypeStruct((_B, _H1), jnp.float32),
            jax.ShapeDtypeStruct((1, _H1), jnp.float32),
            jax.ShapeDtypeStruct((1, _H1), jnp.float32),
        ],
    )(t, a, genre, audio, temporal,
      genre_W, row2(genre_b), audio_W, row2(audio_b),
      temporal_W, row2(temporal_b),
      w1t, w1a, w1g, w1au, w1te, row2(b1))

    y2, s2, ss2 = pl.pallas_call(
        _stage2_body,
        grid=(nb,),
        in_specs=[
            _row_spec(_BM, _H1), _full_spec(1, _H1), _full_spec(1, _H1),
            _full_spec(1, _H1), _full_spec(1, _H1),
            _full_spec(_H1, _H2), _full_spec(1, _H2),
        ],
        out_specs=[
            _row_spec(_BM, _H2), _full_spec(1, _H2), _full_spec(1, _H2),
        ],
        out_shape=[
            jax.ShapeDtypeStruct((_B, _H2), jnp.float32),
            jax.ShapeDtypeStruct((1, _H2), jnp.float32),
            jax.ShapeDtypeStruct((1, _H2), jnp.float32),
        ],
    )(y1, s1, ss1, row2(g1), row2(be1), W2, row2(b2))

    out = pl.pallas_call(
        _stage3_body,
        grid=(nb,),
        in_specs=[
            _row_spec(_BM, _H2), _full_spec(1, _H2), _full_spec(1, _H2),
            _full_spec(1, _H2), _full_spec(1, _H2),
        ],
        out_specs=_row_spec(_BM, _H2),
        out_shape=jax.ShapeDtypeStruct((_B, _H2), jnp.float32),
    )(y2, s2, ss2, row2(g2), row2(be2))
    return out


def kernel(track_id, artist, genre, audio_features, temporal,
           track_table, artist_table,
           genre_W, genre_b, audio_W, audio_b, temporal_W, temporal_b,
           W1, b1, g1, be1, W2, b2, g2, be2):
    t, a = _make_sc_gather()(track_id.astype(jnp.int32),
                             artist.astype(jnp.int32),
                             track_table, artist_table)
    return _tower(t, a, genre, audio_features, temporal,
                  genre_W, genre_b, audio_W, audio_b, temporal_W, temporal_b,
                  W1, b1, g1, be1, W2, b2, g2, be2)


# speedup vs baseline: 1.4471x; 1.4471x over previous
import functools

import jax
import jax.numpy as jnp
from jax import lax
from jax.experimental import pallas as pl
from jax.experimental.pallas import tpu as pltpu
from jax.experimental.pallas import tpu_sc as plsc

_B = 16384
_D = 64
_H1 = 256
_H2 = 128
_EPS = 1e-5
_BM = 2048


@functools.lru_cache(maxsize=1)
def _make_sc_gather():
    info = plsc.get_sparse_core_info()
    nc, ns = info.num_cores, info.num_subcores
    nw = nc * ns
    bpw = _B // nw
    mesh = plsc.VectorSubcoreMesh(core_axis_name="c", subcore_axis_name="s")

    @functools.partial(
        pl.kernel,
        mesh=mesh,
        compiler_params=pltpu.CompilerParams(needs_layout_passes=False),
        out_type=[
            jax.ShapeDtypeStruct((_B, _D), jnp.float32),
            jax.ShapeDtypeStruct((_B, _D), jnp.float32),
        ],
        scratch_types=[
            pltpu.VMEM((bpw,), jnp.int32),
            pltpu.VMEM((bpw, _D), jnp.float32),
            pltpu.SemaphoreType.DMA,
            pltpu.SemaphoreType.DMA,
            pltpu.SemaphoreType.DMA,
        ],
    )
    def gather_k(tid_hbm, aid_hbm, ttab_hbm, atab_hbm, t_out, a_out,
                 ids_v, rows_v, gsem, wsem_t, wsem_a):
        wid = lax.axis_index("s") * nc + lax.axis_index("c")
        base = wid * bpw
        lane = lax.iota(jnp.int32, 16)

        def gather_table(id_hbm, tab_hbm, out_hbm, wsem):
            pltpu.sync_copy(id_hbm.at[pl.ds(base, bpw)], ids_v)

            def fire16(g, carry):
                ids16 = ids_v[pl.ds(g * 16, 16)]
                for j in range(16):
                    rid = jnp.sum(jnp.where(lane == j, ids16, 0))
                    blk = lax.shift_right_logical(rid, 3)
                    sub = rid & 7
                    pltpu.async_copy(
                        tab_hbm.at[blk, sub], rows_v.at[g * 16 + j], gsem)
                return carry

            lax.fori_loop(0, bpw // 16, fire16, 0)

            def drain(i, carry):
                pltpu.make_async_copy(
                    tab_hbm.at[0, 0], rows_v.at[0], gsem).wait()
                return carry

            lax.fori_loop(0, bpw, drain, 0)
            return pltpu.async_copy(rows_v, out_hbm.at[pl.ds(base, bpw)], wsem)

        wt = gather_table(tid_hbm, ttab_hbm, t_out, wsem_t)
        wt.wait()
        wa = gather_table(aid_hbm, atab_hbm, a_out, wsem_a)
        wa.wait()

    return gather_k


def _dot_t(x_t, w):
    return lax.dot_general(x_t[...], w[...], (((0,), (0,)), ((), ())),
                           preferred_element_type=jnp.float32)


def _stage1_body(t, a, genre_t, audio_t, temporal_t,
                 gw, gb, auw, aub, tew, teb,
                 w1t, w1a, w1g, w1au, w1te, b1,
                 y1, s1, ss1):
    i = pl.program_id(0)
    g = jax.nn.relu(_dot_t(genre_t, gw) + gb[...])
    au = jax.nn.relu(_dot_t(audio_t, auw) + aub[...])
    te = jax.nn.relu(_dot_t(temporal_t, tew) + teb[...])
    y = (jnp.dot(t[...], w1t[...], preferred_element_type=jnp.float32)
         + jnp.dot(a[...], w1a[...], preferred_element_type=jnp.float32)
         + jnp.dot(g, w1g[...], preferred_element_type=jnp.float32)
         + jnp.dot(au, w1au[...], preferred_element_type=jnp.float32)
         + jnp.dot(te, w1te[...], preferred_element_type=jnp.float32)
         + b1[...])
    y1[...] = y

    @pl.when(i == 0)
    def _():
        s1[...] = jnp.zeros_like(s1)
        ss1[...] = jnp.zeros_like(ss1)

    s1[...] += jnp.sum(y, axis=0, keepdims=True)
    ss1[...] += jnp.sum(y * y, axis=0, keepdims=True)


def _stage2_body(y1, s1, ss1, g1, be1, w2, b2, y2, s2, ss2):
    i = pl.program_id(0)
    mu = s1[...] * (1.0 / _B)
    var = ss1[...] * (1.0 / _B) - mu * mu
    scale = g1[...] * lax.rsqrt(var + _EPS)
    h = jax.nn.relu((y1[...] - mu) * scale + be1[...])
    y = jnp.dot(h, w2[...], preferred_element_type=jnp.float32) + b2[...]
    y2[...] = y

    @pl.when(i == 0)
    def _():
        s2[...] = jnp.zeros_like(s2)
        ss2[...] = jnp.zeros_like(ss2)

    s2[...] += jnp.sum(y, axis=0, keepdims=True)
    ss2[...] += jnp.sum(y * y, axis=0, keepdims=True)


def _stage3_body(y2, s2, ss2, g2, be2, out):
    mu = s2[...] * (1.0 / _B)
    var = ss2[...] * (1.0 / _B) - mu * mu
    scale = g2[...] * lax.rsqrt(var + _EPS)
    out[...] = jax.nn.relu((y2[...] - mu) * scale + be2[...])


def _row_spec(bm, n):
    return pl.BlockSpec((bm, n), lambda i: (i, 0))


def _full_spec(m, n):
    return pl.BlockSpec((m, n), lambda i: (0, 0))


def _tower(t, a, genre, audio, temporal,
           genre_W, genre_b, audio_W, audio_b, temporal_W, temporal_b,
           W1, b1, g1, be1, W2, b2, g2, be2):
    nb = _B // _BM
    w1t, w1a, w1g, w1au, w1te = (W1[0:64], W1[64:128], W1[128:192],
                                 W1[192:256], W1[256:320])
    row2 = lambda v: v.reshape(1, -1)

    y1, s1, ss1 = pl.pallas_call(
        _stage1_body,
        grid=(nb,),
        in_specs=[
            _row_spec(_BM, _D), _row_spec(_BM, _D),
            pl.BlockSpec((32, _BM), lambda i: (0, i)),
            pl.BlockSpec((16, _BM), lambda i: (0, i)),
            pl.BlockSpec((8, _BM), lambda i: (0, i)),
            _full_spec(32, _D), _full_spec(1, _D),
            _full_spec(16, _D), _full_spec(1, _D),
            _full_spec(8, _D), _full_spec(1, _D),
            _full_spec(_D, _H1), _full_spec(_D, _H1), _full_spec(_D, _H1),
            _full_spec(_D, _H1), _full_spec(_D, _H1), _full_spec(1, _H1),
        ],
        out_specs=[
            _row_spec(_BM, _H1), _full_spec(1, _H1), _full_spec(1, _H1),
        ],
        out_shape=[
            jax.ShapeDtypeStruct((_B, _H1), jnp.float32),
            jax.ShapeDtypeStruct((1, _H1), jnp.float32),
            jax.ShapeDtypeStruct((1, _H1), jnp.float32),
        ],
    )(t, a, genre.T, audio.T, temporal.T,
      genre_W, row2(genre_b), audio_W, row2(audio_b),
      temporal_W, row2(temporal_b),
      w1t, w1a, w1g, w1au, w1te, row2(b1))

    y2, s2, ss2 = pl.pallas_call(
        _stage2_body,
        grid=(nb,),
        in_specs=[
            _row_spec(_BM, _H1), _full_spec(1, _H1), _full_spec(1, _H1),
            _full_spec(1, _H1), _full_spec(1, _H1),
            _full_spec(_H1, _H2), _full_spec(1, _H2),
        ],
        out_specs=[
            _row_spec(_BM, _H2), _full_spec(1, _H2), _full_spec(1, _H2),
        ],
        out_shape=[
            jax.ShapeDtypeStruct((_B, _H2), jnp.float32),
            jax.ShapeDtypeStruct((1, _H2), jnp.float32),
            jax.ShapeDtypeStruct((1, _H2), jnp.float32),
        ],
    )(y1, s1, ss1, row2(g1), row2(be1), W2, row2(b2))

    out = pl.pallas_call(
        _stage3_body,
        grid=(nb,),
        in_specs=[
            _row_spec(_BM, _H2), _full_spec(1, _H2), _full_spec(1, _H2),
            _full_spec(1, _H2), _full_spec(1, _H2),
        ],
        out_specs=_row_spec(_BM, _H2),
        out_shape=jax.ShapeDtypeStruct((_B, _H2), jnp.float32),
    )(y2, s2, ss2, row2(g2), row2(be2))
    return out


def kernel(track_id, artist, genre, audio_features, temporal,
           track_table, artist_table,
           genre_W, genre_b, audio_W, audio_b, temporal_W, temporal_b,
           W1, b1, g1, be1, W2, b2, g2, be2):
    t, a = _make_sc_gather()(track_id.astype(jnp.int32),
                             artist.astype(jnp.int32),
                             track_table.reshape(-1, 8, _D),
                             artist_table.reshape(-1, 8, _D))
    return _tower(t, a, genre, audio_features, temporal,
                  genre_W, genre_b, audio_W, audio_b, temporal_W, temporal_b,
                  W1, b1, g1, be1, W2, b2, g2, be2)


# speedup vs baseline: 1.5326x; 1.0591x over previous
import functools

import jax
import jax.numpy as jnp
from jax import lax
from jax.experimental import pallas as pl
from jax.experimental.pallas import tpu as pltpu
from jax.experimental.pallas import tpu_sc as plsc

_B = 16384
_D = 64
_H1 = 256
_H2 = 128
_EPS = 1e-5
_BM = 2048


@functools.lru_cache(maxsize=1)
def _make_sc_gather():
    info = plsc.get_sparse_core_info()
    nc, ns = info.num_cores, info.num_subcores
    nw = nc * ns
    bpw = _B // nw
    mesh = plsc.VectorSubcoreMesh(core_axis_name="c", subcore_axis_name="s")

    @functools.partial(
        pl.kernel,
        mesh=mesh,
        compiler_params=pltpu.CompilerParams(needs_layout_passes=False),
        out_type=[
            jax.ShapeDtypeStruct((_B, _D), jnp.float32),
            jax.ShapeDtypeStruct((_B, _D), jnp.float32),
        ],
        scratch_types=[
            pltpu.VMEM((bpw,), jnp.int32),
            pltpu.VMEM((bpw, _D), jnp.float32),
            pltpu.SemaphoreType.DMA,
            pltpu.SemaphoreType.DMA,
            pltpu.SemaphoreType.DMA,
        ],
    )
    def gather_k(tid_hbm, aid_hbm, ttab_hbm, atab_hbm, t_out, a_out,
                 ids_v, rows_v, gsem, wsem_t, wsem_a):
        wid = lax.axis_index("s") * nc + lax.axis_index("c")
        base = wid * bpw
        lane = lax.iota(jnp.int32, 16)

        def gather_table(id_hbm, tab_hbm, out_hbm, wsem):
            pltpu.sync_copy(id_hbm.at[pl.ds(base, bpw)], ids_v)

            def fire16(g, carry):
                ids16 = ids_v[pl.ds(g * 16, 16)]
                for j in range(16):
                    rid = jnp.sum(jnp.where(lane == j, ids16, 0))
                    blk = lax.shift_right_logical(rid, 3)
                    sub = rid & 7
                    pltpu.async_copy(
                        tab_hbm.at[blk, sub], rows_v.at[g * 16 + j], gsem)
                return carry

            lax.fori_loop(0, bpw // 16, fire16, 0)

            def drain(i, carry):
                pltpu.make_async_copy(
                    tab_hbm.at[0, 0], rows_v.at[0], gsem).wait()
                return carry

            lax.fori_loop(0, bpw, drain, 0)
            return pltpu.async_copy(rows_v, out_hbm.at[pl.ds(base, bpw)], wsem)

        wt = gather_table(tid_hbm, ttab_hbm, t_out, wsem_t)
        wt.wait()
        wa = gather_table(aid_hbm, atab_hbm, a_out, wsem_a)
        wa.wait()

    return gather_k


def _dot_t(x_t, w):
    return lax.dot_general(x_t[...], w[...], (((0,), (0,)), ((), ())),
                           preferred_element_type=jnp.float32)


def _proj_body(genre_t, audio_t, temporal_t,
               gw, gb, auw, aub, tew, teb,
               w1g, w1au, w1te, b1, p_out):
    g = jax.nn.relu(_dot_t(genre_t, gw) + gb[...])
    au = jax.nn.relu(_dot_t(audio_t, auw) + aub[...])
    te = jax.nn.relu(_dot_t(temporal_t, tew) + teb[...])
    p_out[...] = (jnp.dot(g, w1g[...], preferred_element_type=jnp.float32)
                  + jnp.dot(au, w1au[...], preferred_element_type=jnp.float32)
                  + jnp.dot(te, w1te[...], preferred_element_type=jnp.float32)
                  + b1[...])


def _fused_body(t, a, p, w1t, w1a, g1, be1, w2, b2, g2, be2,
                out, y1_s, y2_s, s1, ss1, s2, ss2):
    ph = pl.program_id(0)
    i = pl.program_id(1)
    rows = pl.ds(i * _BM, _BM)

    @pl.when(jnp.logical_and(ph == 0, i == 0))
    def _():
        s1[...] = jnp.zeros_like(s1)
        ss1[...] = jnp.zeros_like(ss1)
        s2[...] = jnp.zeros_like(s2)
        ss2[...] = jnp.zeros_like(ss2)

    @pl.when(ph == 0)
    def _():
        y = (p[...]
             + jnp.dot(t[...], w1t[...], preferred_element_type=jnp.float32)
             + jnp.dot(a[...], w1a[...], preferred_element_type=jnp.float32))
        y1_s[rows, :] = y
        s1[...] += jnp.sum(y, axis=0, keepdims=True)
        ss1[...] += jnp.sum(y * y, axis=0, keepdims=True)

    @pl.when(ph == 1)
    def _():
        mu = s1[...] * (1.0 / _B)
        var = ss1[...] * (1.0 / _B) - mu * mu
        scale = g1[...] * lax.rsqrt(var + _EPS)
        h = jax.nn.relu((y1_s[rows, :] - mu) * scale + be1[...])
        y = jnp.dot(h, w2[...], preferred_element_type=jnp.float32) + b2[...]
        y2_s[rows, :] = y
        s2[...] += jnp.sum(y, axis=0, keepdims=True)
        ss2[...] += jnp.sum(y * y, axis=0, keepdims=True)

    @pl.when(ph == 2)
    def _():
        mu = s2[...] * (1.0 / _B)
        var = ss2[...] * (1.0 / _B) - mu * mu
        scale = g2[...] * lax.rsqrt(var + _EPS)
        out[...] = jax.nn.relu((y2_s[rows, :] - mu) * scale + be2[...])


def _full_spec2(m, n):
    return pl.BlockSpec((m, n), lambda p, i: (0, 0))


def _tower(t, a, genre, audio, temporal,
           genre_W, genre_b, audio_W, audio_b, temporal_W, temporal_b,
           W1, b1, g1, be1, W2, b2, g2, be2):
    nb = _B // _BM
    w1t, w1a, w1g, w1au, w1te = (W1[0:64], W1[64:128], W1[128:192],
                                 W1[192:256], W1[256:320])
    row2 = lambda v: v.reshape(1, -1)

    p = pl.pallas_call(
        _proj_body,
        grid=(nb,),
        in_specs=[
            pl.BlockSpec((32, _BM), lambda i: (0, i)),
            pl.BlockSpec((16, _BM), lambda i: (0, i)),
            pl.BlockSpec((8, _BM), lambda i: (0, i)),
            pl.BlockSpec((32, _D), lambda i: (0, 0)),
            pl.BlockSpec((1, _D), lambda i: (0, 0)),
            pl.BlockSpec((16, _D), lambda i: (0, 0)),
            pl.BlockSpec((1, _D), lambda i: (0, 0)),
            pl.BlockSpec((8, _D), lambda i: (0, 0)),
            pl.BlockSpec((1, _D), lambda i: (0, 0)),
            pl.BlockSpec((_D, _H1), lambda i: (0, 0)),
            pl.BlockSpec((_D, _H1), lambda i: (0, 0)),
            pl.BlockSpec((_D, _H1), lambda i: (0, 0)),
            pl.BlockSpec((1, _H1), lambda i: (0, 0)),
        ],
        out_specs=pl.BlockSpec((_BM, _H1), lambda i: (i, 0)),
        out_shape=jax.ShapeDtypeStruct((_B, _H1), jnp.float32),
    )(genre.T, audio.T, temporal.T,
      genre_W, row2(genre_b), audio_W, row2(audio_b),
      temporal_W, row2(temporal_b), w1g, w1au, w1te, row2(b1))

    out = pl.pallas_call(
        _fused_body,
        grid=(3, nb),
        in_specs=[
            pl.BlockSpec((_BM, _D), lambda p, i: (jnp.where(p == 0, i, 0), 0)),
            pl.BlockSpec((_BM, _D), lambda p, i: (jnp.where(p == 0, i, 0), 0)),
            pl.BlockSpec((_BM, _H1),
                         lambda p, i: (jnp.where(p == 0, i, 0), 0)),
            _full_spec2(_D, _H1), _full_spec2(_D, _H1),
            _full_spec2(1, _H1), _full_spec2(1, _H1),
            _full_spec2(_H1, _H2), _full_spec2(1, _H2),
            _full_spec2(1, _H2), _full_spec2(1, _H2),
        ],
        out_specs=pl.BlockSpec((_BM, _H2),
                               lambda p, i: (jnp.where(p == 2, i, 0), 0)),
        out_shape=jax.ShapeDtypeStruct((_B, _H2), jnp.float32),
        scratch_shapes=[
            pltpu.VMEM((_B, _H1), jnp.float32),
            pltpu.VMEM((_B, _H2), jnp.float32),
            pltpu.VMEM((1, _H1), jnp.float32),
            pltpu.VMEM((1, _H1), jnp.float32),
            pltpu.VMEM((1, _H2), jnp.float32),
            pltpu.VMEM((1, _H2), jnp.float32),
        ],
        compiler_params=pltpu.CompilerParams(
            vmem_limit_bytes=56 * 1024 * 1024,
            dimension_semantics=("arbitrary", "arbitrary"),
        ),
    )(t, a, p, w1t, w1a, row2(g1), row2(be1), W2, row2(b2),
      row2(g2), row2(be2))
    return out


def kernel(track_id, artist, genre, audio_features, temporal,
           track_table, artist_table,
           genre_W, genre_b, audio_W, audio_b, temporal_W, temporal_b,
           W1, b1, g1, be1, W2, b2, g2, be2):
    t, a = _make_sc_gather()(track_id.astype(jnp.int32),
                             artist.astype(jnp.int32),
                             track_table.reshape(-1, 8, _D),
                             artist_table.reshape(-1, 8, _D))
    return _tower(t, a, genre, audio_features, temporal,
                  genre_W, genre_b, audio_W, audio_b, temporal_W, temporal_b,
                  W1, b1, g1, be1, W2, b2, g2, be2)


# speedup vs baseline: 1.5471x; 1.0094x over previous
import functools

import jax
import jax.numpy as jnp
from jax import lax
from jax.experimental import pallas as pl
from jax.experimental.pallas import tpu as pltpu
from jax.experimental.pallas import tpu_sc as plsc

_B = 16384
_D = 64
_H1 = 256
_H2 = 128
_EPS = 1e-5
_BM = 4096


@functools.lru_cache(maxsize=1)
def _make_sc_gather():
    info = plsc.get_sparse_core_info()
    nc, ns = info.num_cores, info.num_subcores
    nw = nc * ns
    bpw = _B // nw
    mesh = plsc.VectorSubcoreMesh(core_axis_name="c", subcore_axis_name="s")

    @functools.partial(
        pl.kernel,
        mesh=mesh,
        compiler_params=pltpu.CompilerParams(needs_layout_passes=False),
        out_type=[
            jax.ShapeDtypeStruct((_B, _D), jnp.float32),
            jax.ShapeDtypeStruct((_B, _D), jnp.float32),
        ],
        scratch_types=[
            pltpu.VMEM((bpw,), jnp.int32),
            pltpu.VMEM((bpw, _D), jnp.float32),
            pltpu.SemaphoreType.DMA,
            pltpu.SemaphoreType.DMA,
            pltpu.SemaphoreType.DMA,
        ],
    )
    def gather_k(tid_hbm, aid_hbm, ttab_hbm, atab_hbm, t_out, a_out,
                 ids_v, rows_v, gsem, wsem_t, wsem_a):
        wid = lax.axis_index("s") * nc + lax.axis_index("c")
        base = wid * bpw
        lane = lax.iota(jnp.int32, 16)

        def gather_table(id_hbm, tab_hbm, out_hbm, wsem):
            pltpu.sync_copy(id_hbm.at[pl.ds(base, bpw)], ids_v)

            def fire16(g, carry):
                ids16 = ids_v[pl.ds(g * 16, 16)]
                for j in range(16):
                    rid = jnp.sum(jnp.where(lane == j, ids16, 0))
                    blk = lax.shift_right_logical(rid, 3)
                    sub = rid & 7
                    pltpu.async_copy(
                        tab_hbm.at[blk, sub], rows_v.at[g * 16 + j], gsem)
                return carry

            lax.fori_loop(0, bpw // 16, fire16, 0)

            def drain(i, carry):
                pltpu.make_async_copy(
                    tab_hbm.at[0, 0], rows_v.at[0], gsem).wait()
                return carry

            lax.fori_loop(0, bpw, drain, 0)
            return pltpu.async_copy(rows_v, out_hbm.at[pl.ds(base, bpw)], wsem)

        wt = gather_table(tid_hbm, ttab_hbm, t_out, wsem_t)
        wt.wait()
        wa = gather_table(aid_hbm, atab_hbm, a_out, wsem_a)
        wa.wait()

    return gather_k


def _dot_t(x_t, w):
    return lax.dot_general(x_t[...], w[...], (((0,), (0,)), ((), ())),
                           preferred_element_type=jnp.float32)


def _proj_body(genre_t, audio_t, temporal_t,
               gw, gb, auw, aub, tew, teb,
               w1g, w1au, w1te, b1, p_out):
    g = jax.nn.relu(_dot_t(genre_t, gw) + gb[...])
    au = jax.nn.relu(_dot_t(audio_t, auw) + aub[...])
    te = jax.nn.relu(_dot_t(temporal_t, tew) + teb[...])
    p_out[...] = (jnp.dot(g, w1g[...], preferred_element_type=jnp.float32)
                  + jnp.dot(au, w1au[...], preferred_element_type=jnp.float32)
                  + jnp.dot(te, w1te[...], preferred_element_type=jnp.float32)
                  + b1[...])


def _fused_body(t, a, p, w1t, w1a, g1, be1, w2, b2, g2, be2,
                out, y1_s, y2_s, s1, ss1, s2, ss2):
    ph = pl.program_id(0)
    i = pl.program_id(1)
    rows = pl.ds(i * _BM, _BM)

    @pl.when(jnp.logical_and(ph == 0, i == 0))
    def _():
        s1[...] = jnp.zeros_like(s1)
        ss1[...] = jnp.zeros_like(ss1)
        s2[...] = jnp.zeros_like(s2)
        ss2[...] = jnp.zeros_like(ss2)

    @pl.when(ph == 0)
    def _():
        y = (p[...]
             + jnp.dot(t[...], w1t[...], preferred_element_type=jnp.float32)
             + jnp.dot(a[...], w1a[...], preferred_element_type=jnp.float32))
        y1_s[rows, :] = y
        s1[...] += jnp.sum(y, axis=0, keepdims=True)
        ss1[...] += jnp.sum(y * y, axis=0, keepdims=True)

    @pl.when(ph == 1)
    def _():
        mu = s1[...] * (1.0 / _B)
        var = ss1[...] * (1.0 / _B) - mu * mu
        scale = g1[...] * lax.rsqrt(var + _EPS)
        h = jax.nn.relu((y1_s[rows, :] - mu) * scale + be1[...])
        y = jnp.dot(h, w2[...], preferred_element_type=jnp.float32) + b2[...]
        y2_s[rows, :] = y
        s2[...] += jnp.sum(y, axis=0, keepdims=True)
        ss2[...] += jnp.sum(y * y, axis=0, keepdims=True)

    @pl.when(ph == 2)
    def _():
        mu = s2[...] * (1.0 / _B)
        var = ss2[...] * (1.0 / _B) - mu * mu
        scale = g2[...] * lax.rsqrt(var + _EPS)
        out[...] = jax.nn.relu((y2_s[rows, :] - mu) * scale + be2[...])


def _full_spec2(m, n):
    return pl.BlockSpec((m, n), lambda p, i: (0, 0))


def _tower(t, a, genre, audio, temporal,
           genre_W, genre_b, audio_W, audio_b, temporal_W, temporal_b,
           W1, b1, g1, be1, W2, b2, g2, be2):
    nb = _B // _BM
    w1t, w1a, w1g, w1au, w1te = (W1[0:64], W1[64:128], W1[128:192],
                                 W1[192:256], W1[256:320])
    row2 = lambda v: v.reshape(1, -1)

    p = pl.pallas_call(
        _proj_body,
        grid=(nb,),
        in_specs=[
            pl.BlockSpec((32, _BM), lambda i: (0, i)),
            pl.BlockSpec((16, _BM), lambda i: (0, i)),
            pl.BlockSpec((8, _BM), lambda i: (0, i)),
            pl.BlockSpec((32, _D), lambda i: (0, 0)),
            pl.BlockSpec((1, _D), lambda i: (0, 0)),
            pl.BlockSpec((16, _D), lambda i: (0, 0)),
            pl.BlockSpec((1, _D), lambda i: (0, 0)),
            pl.BlockSpec((8, _D), lambda i: (0, 0)),
            pl.BlockSpec((1, _D), lambda i: (0, 0)),
            pl.BlockSpec((_D, _H1), lambda i: (0, 0)),
            pl.BlockSpec((_D, _H1), lambda i: (0, 0)),
            pl.BlockSpec((_D, _H1), lambda i: (0, 0)),
            pl.BlockSpec((1, _H1), lambda i: (0, 0)),
        ],
        out_specs=pl.BlockSpec((_BM, _H1), lambda i: (i, 0)),
        out_shape=jax.ShapeDtypeStruct((_B, _H1), jnp.float32),
    )(genre.T, audio.T, temporal.T,
      genre_W, row2(genre_b), audio_W, row2(audio_b),
      temporal_W, row2(temporal_b), w1g, w1au, w1te, row2(b1))

    out = pl.pallas_call(
        _fused_body,
        grid=(3, nb),
        in_specs=[
            pl.BlockSpec((_BM, _D), lambda p, i: (jnp.where(p == 0, i, 0), 0)),
            pl.BlockSpec((_BM, _D), lambda p, i: (jnp.where(p == 0, i, 0), 0)),
            pl.BlockSpec((_BM, _H1),
                         lambda p, i: (jnp.where(p == 0, i, 0), 0)),
            _full_spec2(_D, _H1), _full_spec2(_D, _H1),
            _full_spec2(1, _H1), _full_spec2(1, _H1),
            _full_spec2(_H1, _H2), _full_spec2(1, _H2),
            _full_spec2(1, _H2), _full_spec2(1, _H2),
        ],
        out_specs=pl.BlockSpec((_BM, _H2),
                               lambda p, i: (jnp.where(p == 2, i, 0), 0)),
        out_shape=jax.ShapeDtypeStruct((_B, _H2), jnp.float32),
        scratch_shapes=[
            pltpu.VMEM((_B, _H1), jnp.float32),
            pltpu.VMEM((_B, _H2), jnp.float32),
            pltpu.VMEM((1, _H1), jnp.float32),
            pltpu.VMEM((1, _H1), jnp.float32),
            pltpu.VMEM((1, _H2), jnp.float32),
            pltpu.VMEM((1, _H2), jnp.float32),
        ],
        compiler_params=pltpu.CompilerParams(
            vmem_limit_bytes=56 * 1024 * 1024,
            dimension_semantics=("arbitrary", "arbitrary"),
        ),
    )(t, a, p, w1t, w1a, row2(g1), row2(be1), W2, row2(b2),
      row2(g2), row2(be2))
    return out


def kernel(track_id, artist, genre, audio_features, temporal,
           track_table, artist_table,
           genre_W, genre_b, audio_W, audio_b, temporal_W, temporal_b,
           W1, b1, g1, be1, W2, b2, g2, be2):
    t, a = _make_sc_gather()(track_id.astype(jnp.int32),
                             artist.astype(jnp.int32),
                             track_table.reshape(-1, 8, _D),
                             artist_table.reshape(-1, 8, _D))
    return _tower(t, a, genre, audio_features, temporal,
                  genre_W, genre_b, audio_W, audio_b, temporal_W, temporal_b,
                  W1, b1, g1, be1, W2, b2, g2, be2)
